# baseline (speedup 1.0000x reference)
import functools

import jax
import jax.numpy as jnp
from jax import lax
from jax.experimental import pallas as pl
from jax.experimental.pallas import tpu as pltpu
from jax.experimental.pallas import tpu_sc as plsc

P_ = 24
H_ = 12
W_ = 25
LANES_ = 32
HL_ = 16
CHUNK_CYC_ = 16
CHUNK_ = CHUNK_CYC_ * P_


def _pass1_strip(S, xbuf, trings, h, pre_chunk, emit_chunk):
    lo = h * HL_
    n_cycles = S // P_
    full_chunks = (n_cycles - 1) // CHUNK_CYC_
    ls = slice(lo, lo + HL_)

    def ring_row(t):
        return t - (t // CHUNK_) * CHUNK_

    pre_chunk(0)
    w = xbuf[0, ls]
    for d in range(1, H_ + 1):
        w = w + xbuf[d, ls]

    dline = []
    for t in range(P_):
        r = 1.0 / (H_ + 1 + t) if t <= H_ else 1.0 / W_
        tr = w * r
        trings[0][ring_row(t), ls] = tr
        x_m = xbuf[t - H_, ls] if t >= H_ else None
        det = xbuf[t, ls] - tr
        dline.append(det)
        if len(dline) > 12:
            xbuf[t - 12, ls] = dline.pop(0)
        w = w + xbuf[t + H_ + 1, ls]
        if t >= H_:
            w = w - x_m

    def mid_body(kk, par):
        base = kk * CHUNK_

        def body(t, carry):
            wc = carry[0]
            dl = list(carry[1:])
            tr = wc * (1.0 / W_)
            trings[par][t - base, ls] = tr
            x_m = xbuf[t - H_, ls]
            det = xbuf[t, ls] - tr
            dl.append(det)
            xbuf[t - 12, ls] = dl.pop(0)
            wc = wc + xbuf[t + H_ + 1, ls] - x_m
            return (wc, *dl)
        return body

    carry = (w, *dline)
    for k in range(full_chunks):
        if k > 0:
            pre_chunk(k)
        t_lo = max(P_, k * CHUNK_)
        t_hi = (k + 1) * CHUNK_
        carry = lax.fori_loop(t_lo, t_hi, mid_body(k, k % 2), carry,
                              unroll=4)
        emit_chunk(k)
    k = full_chunks
    pre_chunk(k)
    carry = lax.fori_loop(k * CHUNK_, (n_cycles - 1) * P_,
                          mid_body(k, k % 2), carry, unroll=4)
    w = carry[0]
    dline = list(carry[1:])

    for t in range((n_cycles - 1) * P_, S):
        r = 1.0 / W_ if t + H_ + 1 <= S else 1.0 / (S - t + H_)
        tr = w * r
        trings[k % 2][ring_row(t), ls] = tr
        x_m = xbuf[t - H_, ls]
        det = xbuf[t, ls] - tr
        dline.append(det)
        xbuf[t - 12, ls] = dline.pop(0)
        if t + H_ + 1 < S:
            w = w + xbuf[t + H_ + 1, ls]
        w = w - x_m
    for i, det in enumerate(dline):
        xbuf[S - 12 + i, ls] = det
    emit_chunk(k)


def _phase_sums(S, xbuf, h):
    lo = h * HL_
    ls = slice(lo, lo + HL_)
    n_cycles = S // P_
    rem = S % P_
    psum = []
    for p in range(P_):
        n = n_cycles + 1 if p < rem else n_cycles

        def body(c, acc, _p=p):
            return acc + xbuf[c * P_ + _p, ls]

        acc = lax.fori_loop(1, n, body, xbuf[p, ls], unroll=5)
        psum.append(acc)
    return psum


def _pass2_strip(S, xbuf, sbuf, h, pat):
    lo = h * HL_
    ls = slice(lo, lo + HL_)
    n_cycles = S // P_

    def sfill(c, dummy):
        base = c * P_
        for p in range(P_):
            sbuf[base + p, ls] = pat[p]
        return dummy

    lax.fori_loop(0, CHUNK_CYC_, sfill, jnp.int32(0), unroll=False)

    def rbody(c, dummy):
        base = c * P_
        for p in range(P_):
            t = base + p
            xbuf[t, ls] = xbuf[t, ls] - pat[p]
        return dummy

    lax.fori_loop(0, n_cycles, rbody, jnp.int32(0), unroll=False)
    for t in range(n_cycles * P_, S):
        xbuf[t, ls] = xbuf[t, ls] - pat[t % P_]


def _decomp_body(S, B, F, NT, x_hbm, trend_hbm, seasonal_hbm, residual_hbm,
                 xbuf, tr0, tr1, sbuf,
                 sem_in, sem_t0, sem_t1, sem_s, sem_r):
    n_cycles = S // P_
    rem = S % P_
    info = plsc.get_sparse_core_info()
    nc = info.num_cores
    groups = F // LANES_
    wid = lax.axis_index("s") * nc + lax.axis_index("c")

    trings = [tr0, tr1]
    sems_t = [sem_t0, sem_t1]
    n_chunks = (S + CHUNK_ - 1) // CHUNK_

    def loc(j):
        task = wid * NT + j
        return task // groups, (task % groups) * LANES_

    state = {"t_h": [None, None], "s_h": [], "r_h": None}

    b0, l0 = loc(0)
    in_h = pltpu.async_copy(x_hbm.at[b0, :, pl.ds(l0, LANES_)], xbuf, sem_in)

    for j in range(NT):
        b, l = loc(j)
        in_h.wait()
        for h in range(2):
            def pre_chunk(k):
                par = k % 2
                if state["t_h"][par] is not None:
                    state["t_h"][par].wait()
                    state["t_h"][par] = None

            def emit_chunk(k, _h=h, _b=b, _l=l):
                par = k % 2
                rows = min(CHUNK_, S - k * CHUNK_)
                state["t_h"][par] = pltpu.async_copy(
                    trings[par].at[pl.ds(0, rows), pl.ds(_h * HL_, HL_)],
                    trend_hbm.at[_b, pl.ds(k * CHUNK_, rows),
                                 pl.ds(_l + _h * HL_, HL_)],
                    sems_t[par])

            _pass1_strip(S, xbuf, trings, h, pre_chunk, emit_chunk)
            psum = _phase_sums(S, xbuf, h)
            pat = [psum[p] * (1.0 / (n_cycles + 1 if p < rem else n_cycles))
                   for p in range(P_)]
            if h == 0 and state["s_h"]:
                for c in state["s_h"]:
                    c.wait()
                state["s_h"] = []
            _pass2_strip(S, xbuf, sbuf, h, pat)

        state["r_h"] = pltpu.async_copy(
            xbuf, residual_hbm.at[b, :, pl.ds(l, LANES_)], sem_r)
        off = 0
        while off < S:
            rows = min(CHUNK_, S - off)
            state["s_h"].append(pltpu.async_copy(
                sbuf.at[pl.ds(0, rows)],
                seasonal_hbm.at[b, pl.ds(off, rows), pl.ds(l, LANES_)],
                sem_s))
            off += rows
        if j + 1 < NT:
            nb, nl = loc(j + 1)
            state["r_h"].wait()
            state["r_h"] = None
            in_h = pltpu.async_copy(x_hbm.at[nb, :, pl.ds(nl, LANES_)],
                                    xbuf, sem_in)

    for c in state["t_h"]:
        if c is not None:
            c.wait()
    for c in state["s_h"]:
        c.wait()
    if state["r_h"] is not None:
        state["r_h"].wait()


@jax.jit
def _decompose(x):
    B, S, F = x.shape
    info = plsc.get_sparse_core_info()
    n_workers = info.num_cores * info.num_subcores
    n_tasks = B * (F // LANES_)
    assert n_tasks % n_workers == 0
    mesh = plsc.VectorSubcoreMesh(core_axis_name="c", subcore_axis_name="s")
    out = jax.ShapeDtypeStruct((B, S, F), x.dtype)
    body = functools.partial(_decomp_body, S, B, F, n_tasks // n_workers)
    return pl.kernel(
        body,
        out_type=(out, out, out),
        mesh=mesh,
        scratch_types=[
            pltpu.VMEM((S, LANES_), jnp.float32),
            pltpu.VMEM((CHUNK_, LANES_), jnp.float32),
            pltpu.VMEM((CHUNK_, LANES_), jnp.float32),
            pltpu.VMEM((CHUNK_, LANES_), jnp.float32),
            pltpu.SemaphoreType.DMA,
            pltpu.SemaphoreType.DMA,
            pltpu.SemaphoreType.DMA,
            pltpu.SemaphoreType.DMA,
            pltpu.SemaphoreType.DMA,
        ],
        compiler_params=pltpu.CompilerParams(use_tc_tiling_on_sc=False),
    )(x)


def kernel(x):
    trend, seasonal, residual = _decompose(x)
    return (trend, seasonal, residual, x)
